# Initial kernel scaffold; baseline (speedup 1.0000x reference)
import jax
import jax.numpy as jnp
from jax.experimental import pallas as pl


def kernel(encoding, available_task):
    raise NotImplementedError("write your pallas kernel here")


# speedup vs baseline: 1.9712x; 1.9712x over previous
import functools

import jax
import jax.numpy as jnp
from jax import lax
from jax.experimental import pallas as pl
from jax.experimental.pallas import tpu as pltpu
from jax.experimental.pallas import tpu_sc as plsc

_NC = 2
_NS = 16
_NW = _NC * _NS


@functools.partial(jax.jit, static_argnames=("b_per_w", "chunk", "d"))
def _sc_gather(table, idx, *, b_per_w, chunk, d):
    batch = idx.shape[0]
    nchunk = b_per_w // chunk
    mesh = plsc.VectorSubcoreMesh(core_axis_name="c", subcore_axis_name="s")

    @functools.partial(
        pl.kernel,
        out_type=jax.ShapeDtypeStruct((batch, d), jnp.float32),
        mesh=mesh,
        scratch_types=[
            pltpu.VMEM((b_per_w,), jnp.int32),
            pltpu.VMEM((chunk, d), jnp.float32),
            pltpu.SemaphoreType.DMA,
        ],
    )
    def k(table_hbm, idx_hbm, out_hbm, idx_v, rows_v, sem):
        wid = lax.axis_index("s") * _NC + lax.axis_index("c")
        base = wid * b_per_w
        pltpu.sync_copy(idx_hbm.at[pl.ds(base, b_per_w)], idx_v)

        def body(c, _):
            g = pltpu.async_copy(
                table_hbm.at[idx_v.at[pl.ds(c * chunk, chunk)]], rows_v, sem
            )
            g.wait()
            pltpu.sync_copy(rows_v, out_hbm.at[pl.ds(base + c * chunk, chunk)])
            return 0

        lax.fori_loop(0, nchunk, body, 0)

    return k(table, idx)


def kernel(encoding, available_task):
    _, task_num, d = encoding.shape
    table = encoding.reshape(task_num, d)
    idx = available_task.astype(jnp.int32)
    batch = idx.shape[0]
    out = _sc_gather(table, idx, b_per_w=batch // _NW, chunk=64, d=d)
    return out.reshape(1, batch, d)


# speedup vs baseline: 2.1214x; 1.0762x over previous
import functools

import jax
import jax.numpy as jnp
from jax import lax
from jax.experimental import pallas as pl
from jax.experimental.pallas import tpu as pltpu
from jax.experimental.pallas import tpu_sc as plsc

_NC = 2
_NS = 16
_NW = _NC * _NS


@functools.partial(jax.jit, static_argnames=("b_per_w", "chunk", "nbuf", "d"))
def _sc_gather(table, idx, *, b_per_w, chunk, nbuf, d):
    batch = idx.shape[0]
    nchunk = b_per_w // chunk
    nstep = nchunk // nbuf
    mesh = plsc.VectorSubcoreMesh(core_axis_name="c", subcore_axis_name="s")

    @functools.partial(
        pl.kernel,
        out_type=jax.ShapeDtypeStruct((batch, d), jnp.float32),
        mesh=mesh,
        scratch_types=[
            pltpu.VMEM((b_per_w,), jnp.int32),
            [pltpu.VMEM((chunk, d), jnp.float32)] * nbuf,
            [pltpu.SemaphoreType.DMA] * nbuf,
            [pltpu.SemaphoreType.DMA] * nbuf,
        ],
    )
    def k(table_hbm, idx_hbm, out_hbm, idx_v, bufs, gsems, wsems):
        wid = lax.axis_index("s") * _NC + lax.axis_index("c")
        base = wid * b_per_w
        pltpu.sync_copy(idx_hbm.at[pl.ds(base, b_per_w)], idx_v)

        def gather(c, b):
            pltpu.async_copy(
                table_hbm.at[idx_v.at[pl.ds(c * chunk, chunk)]], bufs[b], gsems[b]
            )

        def gwait(b):
            pltpu.make_async_copy(
                table_hbm.at[pl.ds(0, chunk)], bufs[b], gsems[b]
            ).wait()

        def write(c, b):
            pltpu.async_copy(
                bufs[b], out_hbm.at[pl.ds(base + c * chunk, chunk)], wsems[b]
            )

        def wwait(b):
            pltpu.make_async_copy(
                bufs[b], out_hbm.at[pl.ds(0, chunk)], wsems[b]
            ).wait()

        for b in range(nbuf):
            gather(b, b)

        def body(p, _):
            for b in range(nbuf):
                c = p * nbuf + b
                gwait(b)
                write(c, b)

                @pl.when(p < nstep - 1)
                def _():
                    wwait(b)
                    gather(c + nbuf, b)

            return 0

        lax.fori_loop(0, nstep, body, 0)
        for b in range(nbuf):
            wwait(b)

    return k(table, idx)


def kernel(encoding, available_task):
    _, task_num, d = encoding.shape
    table = encoding.reshape(task_num, d)
    idx = available_task.astype(jnp.int32)
    batch = idx.shape[0]
    out = _sc_gather(table, idx, b_per_w=batch // _NW, chunk=16, nbuf=4, d=d)
    return out.reshape(1, batch, d)
